# baseline (speedup 1.0000x reference)
import functools

import numpy as np
import jax
import jax.numpy as jnp
from jax import lax
from jax.experimental import pallas as pl
from jax.experimental.pallas import tpu as pltpu
from jax.experimental.pallas import tpu_sc as plsc

_T, _B, _C = 1024, 64, 192
_RATIO = 0.75
_REMAIN_T = int(_T * (1 - _RATIO))

_NC, _NS = 2, 16
_NW = _NC * _NS
_CS = 96
_CBLK = 32
_NCB = _CS // _CBLK
_NTASK = _B * _NCB
_TPW = _NTASK // _NW
_L = 16


def _host_indexes():
    side = int(_T**0.5)
    mask_t = side * side - _REMAIN_T
    block_side = int(mask_t**0.5)
    rng = np.random.RandomState(0)
    fwd, bwd = [], []
    for _ in range(_B):
        i = rng.randint(0, side - block_side + 1)
        j = rng.randint(0, side - block_side + 1)
        mask = np.zeros((side, side), dtype=np.float32)
        mask[i : i + block_side, j : j + block_side] = 1
        mask = mask.flatten()
        f = np.where(mask == 0)[0]
        b = np.argsort(np.concatenate((f, np.where(mask == 1)[0])))
        fwd.append(f)
        bwd.append(b)
    forward = np.stack(fwd, axis=-1).astype(np.int32)
    backward = np.stack(bwd, axis=-1).astype(np.int32)
    return forward, backward


_FWD_NP, _BWD_NP = _host_indexes()
_IDXT_NP = np.ascontiguousarray(_FWD_NP[:_REMAIN_T].T).astype(np.int32)


@functools.cache
def _build_sc_gather():
    @functools.partial(
        pl.kernel,
        out_type=jax.ShapeDtypeStruct((_B, _CS, _REMAIN_T), jnp.float32),
        mesh=plsc.VectorSubcoreMesh(
            core_axis_name="c", subcore_axis_name="s", num_cores=_NC, num_subcores=_NS
        ),
        scratch_types=[
            pltpu.VMEM((2, _CBLK, _T), jnp.float32),
            pltpu.VMEM((2, _CBLK, _REMAIN_T), jnp.float32),
            pltpu.VMEM((2, _REMAIN_T), jnp.int32),
            pltpu.SemaphoreType.DMA,
            pltpu.SemaphoreType.DMA,
        ],
        compiler_params=pltpu.CompilerParams(needs_layout_passes=False),
    )
    def _sc_gather(pt_hbm, idx_hbm, out_hbm, inbuf, outbuf, idx_v, sem_g, sem_w):
        wid = lax.axis_index("s") * _NC + lax.axis_index("c")
        task0 = wid * _TPW

        def stage(k, s):
            tk = task0 + k
            b, cb = tk // _NCB, tk % _NCB
            return (
                pltpu.async_copy(
                    pt_hbm.at[pl.ds(b, 1), pl.ds(cb * _CBLK, _CBLK)],
                    inbuf.at[pl.ds(s, 1)],
                    sem_g,
                ),
                pltpu.async_copy(
                    idx_hbm.at[pl.ds(b, 1)], idx_v.at[pl.ds(s, 1)], sem_g
                ),
            )

        def write(k, s):
            tk = task0 + k
            b, cb = tk // _NCB, tk % _NCB
            return pltpu.async_copy(
                outbuf.at[pl.ds(s, 1)],
                out_hbm.at[pl.ds(b, 1), pl.ds(cb * _CBLK, _CBLK)],
                sem_w,
            )

        def compute(s):
            slot = jnp.full((_L,), s, jnp.int32)

            def jbody(j, _):
                col = idx_v[s, pl.ds(j * _L, _L)]
                for c in range(_CBLK):
                    row = jnp.full((_L,), c, jnp.int32)
                    outbuf[s, c, pl.ds(j * _L, _L)] = plsc.load_gather(
                        inbuf, [slot, row, col]
                    )
                return _

            lax.fori_loop(0, _REMAIN_T // _L, jbody, 0)

        g = [None] * _TPW
        w = [None] * _TPW
        g[0] = stage(0, 0)
        for k in range(_TPW):
            s = k % 2
            if k >= 2:
                w[k - 2].wait()
            if k + 1 < _TPW:
                g[k + 1] = stage(k + 1, 1 - s)
            for cp in g[k]:
                cp.wait()
            compute(s)
            w[k] = write(k, s)
        w[_TPW - 2].wait()
        w[_TPW - 1].wait()

    return _sc_gather


def _tc_tail_body(pt_ref, idx_ref, out_ref):
    idx = idx_ref[0, 0, :]
    token = jax.lax.broadcasted_iota(jnp.int32, (_T, _REMAIN_T), 0)
    onehot = (token == idx[None, :]).astype(jnp.float32)
    out_ref[0] = jnp.dot(pt_ref[0], onehot, preferred_element_type=jnp.float32)


@functools.cache
def _build_tc_tail():
    return pl.pallas_call(
        _tc_tail_body,
        out_shape=jax.ShapeDtypeStruct((_B, _C - _CS, _REMAIN_T), jnp.float32),
        grid=(_B,),
        in_specs=[
            pl.BlockSpec((1, _C - _CS, _T), lambda b: (b, 1, 0)),
            pl.BlockSpec((1, 1, _REMAIN_T), lambda b: (b, 0, 0)),
        ],
        out_specs=pl.BlockSpec((1, _C - _CS, _REMAIN_T), lambda b: (b, 0, 0)),
    )


def kernel(patches):
    pt = lax.transpose(patches, (1, 2, 0))
    idxt = jnp.asarray(_IDXT_NP)
    head_pt = _build_sc_gather()(pt, idxt)
    tail_pt = _build_tc_tail()(pt, idxt[:, None, :])
    out_pt = jnp.concatenate([head_pt, tail_pt], axis=1)
    masked = lax.transpose(out_pt, (2, 0, 1))
    fwd = lax.transpose(jnp.asarray(np.ascontiguousarray(_FWD_NP.T)), (1, 0))
    bwd = lax.transpose(jnp.asarray(np.ascontiguousarray(_BWD_NP.T)), (1, 0))
    return masked, fwd, bwd


# speedup vs baseline: 1.3603x; 1.3603x over previous
import functools

import numpy as np
import jax
import jax.numpy as jnp
from jax import lax
from jax.experimental import pallas as pl
from jax.experimental.pallas import tpu as pltpu
from jax.experimental.pallas import tpu_sc as plsc

_T, _B, _C = 1024, 64, 192
_RATIO = 0.75
_REMAIN_T = int(_T * (1 - _RATIO))

_NC, _NS = 2, 16
_NW = _NC * _NS
_CBLK = 48
_NCB = _C // _CBLK
_NTASK = _B * _NCB
_TPW = _NTASK // _NW
_L = 16


def _host_indexes():
    side = int(_T**0.5)
    mask_t = side * side - _REMAIN_T
    block_side = int(mask_t**0.5)
    rng = np.random.RandomState(0)
    fwd, bwd = [], []
    for _ in range(_B):
        i = rng.randint(0, side - block_side + 1)
        j = rng.randint(0, side - block_side + 1)
        mask = np.zeros((side, side), dtype=np.float32)
        mask[i : i + block_side, j : j + block_side] = 1
        mask = mask.flatten()
        f = np.where(mask == 0)[0]
        b = np.argsort(np.concatenate((f, np.where(mask == 1)[0])))
        fwd.append(f)
        bwd.append(b)
    forward = np.stack(fwd, axis=-1).astype(np.int32)
    backward = np.stack(bwd, axis=-1).astype(np.int32)
    return forward, backward


_FWD_NP, _BWD_NP = _host_indexes()
_IDXT_NP = np.ascontiguousarray(_FWD_NP[:_REMAIN_T].T).astype(np.int32)


@functools.cache
def _build_sc_gather():
    @functools.partial(
        pl.kernel,
        out_type=jax.ShapeDtypeStruct((_B, _C, _REMAIN_T), jnp.float32),
        mesh=plsc.VectorSubcoreMesh(
            core_axis_name="c", subcore_axis_name="s", num_cores=_NC, num_subcores=_NS
        ),
        scratch_types=[
            pltpu.VMEM((2, _CBLK, _T), jnp.float32),
            pltpu.VMEM((2, _CBLK, _REMAIN_T), jnp.float32),
            pltpu.VMEM((2, _REMAIN_T), jnp.int32),
            pltpu.SemaphoreType.DMA,
            pltpu.SemaphoreType.DMA,
        ],
        compiler_params=pltpu.CompilerParams(needs_layout_passes=False),
    )
    def _sc_gather(pt_hbm, idx_hbm, out_hbm, inbuf, outbuf, idx_v, sem_g, sem_w):
        wid = lax.axis_index("s") * _NC + lax.axis_index("c")
        task0 = wid * _TPW

        def stage(k, s):
            tk = task0 + k
            b, cb = tk // _NCB, tk % _NCB
            return (
                pltpu.async_copy(
                    pt_hbm.at[pl.ds(b, 1), pl.ds(cb * _CBLK, _CBLK)],
                    inbuf.at[pl.ds(s, 1)],
                    sem_g,
                ),
                pltpu.async_copy(
                    idx_hbm.at[pl.ds(b, 1)], idx_v.at[pl.ds(s, 1)], sem_g
                ),
            )

        def write(k, s):
            tk = task0 + k
            b, cb = tk // _NCB, tk % _NCB
            return pltpu.async_copy(
                outbuf.at[pl.ds(s, 1)],
                out_hbm.at[pl.ds(b, 1), pl.ds(cb * _CBLK, _CBLK)],
                sem_w,
            )

        def compute(s):
            slot = jnp.full((_L,), s, jnp.int32)

            def jbody(j, _):
                col = idx_v[s, pl.ds(j * _L, _L)]
                for c in range(_CBLK):
                    row = jnp.full((_L,), c, jnp.int32)
                    outbuf[s, c, pl.ds(j * _L, _L)] = plsc.load_gather(
                        inbuf, [slot, row, col]
                    )
                return _

            lax.fori_loop(0, _REMAIN_T // _L, jbody, 0)

        g = [None] * _TPW
        w = [None] * _TPW
        g[0] = stage(0, 0)
        for k in range(_TPW):
            s = k % 2
            if k >= 2:
                w[k - 2].wait()
            if k + 1 < _TPW:
                g[k + 1] = stage(k + 1, 1 - s)
            for cp in g[k]:
                cp.wait()
            compute(s)
            w[k] = write(k, s)
        w[_TPW - 2].wait()
        w[_TPW - 1].wait()

    return _sc_gather


def kernel(patches):
    pt = lax.transpose(patches, (1, 2, 0))
    out_pt = _build_sc_gather()(pt, jnp.asarray(_IDXT_NP))
    masked = lax.transpose(out_pt, (2, 0, 1))
    fwd = lax.transpose(jnp.asarray(np.ascontiguousarray(_FWD_NP.T)), (1, 0))
    bwd = lax.transpose(jnp.asarray(np.ascontiguousarray(_BWD_NP.T)), (1, 0))
    return masked, fwd, bwd
